# baseline (speedup 1.0000x reference)
import functools

import jax
import jax.numpy as jnp
from jax import lax
from jax.experimental import pallas as pl
from jax.experimental.pallas import tpu as pltpu
from jax.experimental.pallas import tpu_sc as plsc

_ROWS = 1000
_EMBED_DIM = 128
_BATCH = 16384

_info = plsc.get_sparse_core_info()
_NC, _NS = _info.num_cores, _info.num_subcores
_NW = _NC * _NS
_B_PER_W = _BATCH // _NW
_CHUNKS = (32, 64, 128, 128, 128, 32)
_NCH = len(_CHUNKS)
_OFFS = tuple(sum(_CHUNKS[:i]) for i in range(_NCH))
_ROWS_PER_TILE = 64

_mesh = plsc.VectorSubcoreMesh(core_axis_name="c", subcore_axis_name="s")


@functools.partial(
    pl.kernel,
    mesh=_mesh,
    out_type=jax.ShapeDtypeStruct((_BATCH, _EMBED_DIM), jnp.float32),
    scratch_types=[
        pltpu.VMEM((_B_PER_W,), jnp.int32),
        pltpu.VMEM((_B_PER_W, _EMBED_DIM), jnp.float32),
        pltpu.VMEM_SHARED((_ROWS, _EMBED_DIM), jnp.float32),
    ]
    + [pltpu.SemaphoreType.DMA] * (_NCH + 2),
)
def _gather_kernel(idx_hbm, table_hbm, out_hbm, idx_v, rows_v, table_sh, *sems):
    gsems, wsem, isem = sems[:_NCH], sems[_NCH], sems[_NCH + 1]
    sid = lax.axis_index("s")
    wid = sid * _NC + lax.axis_index("c")
    base = wid * _B_PER_W
    icp = pltpu.async_copy(idx_hbm.at[pl.ds(base, _B_PER_W)], idx_v, isem)
    r0 = jnp.minimum(sid * _ROWS_PER_TILE, _ROWS - _ROWS_PER_TILE)
    pltpu.sync_copy(table_hbm.at[pl.ds(r0, _ROWS_PER_TILE)],
                    table_sh.at[pl.ds(r0, _ROWS_PER_TILE)])
    icp.wait()
    plsc.subcore_barrier()
    rds = [
        pltpu.async_copy(table_sh.at[idx_v.at[pl.ds(_OFFS[c], _CHUNKS[c])]],
                         rows_v.at[pl.ds(_OFFS[c], _CHUNKS[c])], gsems[c])
        for c in range(_NCH)
    ]
    wrs = []
    for c in range(_NCH):
        rds[c].wait()
        wrs.append(
            pltpu.async_copy(rows_v.at[pl.ds(_OFFS[c], _CHUNKS[c])],
                             out_hbm.at[pl.ds(base + _OFFS[c], _CHUNKS[c])], wsem)
        )
    for w in wrs:
        w.wait()


def kernel(t, embedding_weight):
    return _gather_kernel(t.astype(jnp.int32), embedding_weight)


# speedup vs baseline: 1.0102x; 1.0102x over previous
import functools

import jax
import jax.numpy as jnp
from jax import lax
from jax.experimental import pallas as pl
from jax.experimental.pallas import tpu as pltpu
from jax.experimental.pallas import tpu_sc as plsc

_ROWS = 1000
_EMBED_DIM = 128
_BATCH = 16384

_info = plsc.get_sparse_core_info()
_NC, _NS = _info.num_cores, _info.num_subcores
_NW = _NC * _NS
_B_PER_W = _BATCH // _NW
_CHUNKS = (64, 128, 128, 160, 32)
_NCH = len(_CHUNKS)
_OFFS = tuple(sum(_CHUNKS[:i]) for i in range(_NCH))
_ROWS_PER_TILE = 64

_mesh = plsc.VectorSubcoreMesh(core_axis_name="c", subcore_axis_name="s")


@functools.partial(
    pl.kernel,
    mesh=_mesh,
    out_type=jax.ShapeDtypeStruct((_BATCH, _EMBED_DIM), jnp.float32),
    scratch_types=[
        pltpu.VMEM((_B_PER_W,), jnp.int32),
        pltpu.VMEM((_B_PER_W, _EMBED_DIM), jnp.float32),
        pltpu.VMEM_SHARED((_ROWS, _EMBED_DIM), jnp.float32),
    ]
    + [pltpu.SemaphoreType.DMA] * (_NCH + 2),
)
def _gather_kernel(idx_hbm, table_hbm, out_hbm, idx_v, rows_v, table_sh, *sems):
    gsems, wsem, isem = sems[:_NCH], sems[_NCH], sems[_NCH + 1]
    sid = lax.axis_index("s")
    wid = sid * _NC + lax.axis_index("c")
    base = wid * _B_PER_W
    icp = pltpu.async_copy(idx_hbm.at[pl.ds(base, _B_PER_W)], idx_v, isem)
    r0 = jnp.minimum(sid * _ROWS_PER_TILE, _ROWS - _ROWS_PER_TILE)
    pltpu.sync_copy(table_hbm.at[pl.ds(r0, _ROWS_PER_TILE)],
                    table_sh.at[pl.ds(r0, _ROWS_PER_TILE)])
    icp.wait()
    plsc.subcore_barrier()
    rds = [
        pltpu.async_copy(table_sh.at[idx_v.at[pl.ds(_OFFS[c], _CHUNKS[c])]],
                         rows_v.at[pl.ds(_OFFS[c], _CHUNKS[c])], gsems[c])
        for c in range(_NCH)
    ]
    wrs = []
    for c in range(_NCH):
        rds[c].wait()
        wrs.append(
            pltpu.async_copy(rows_v.at[pl.ds(_OFFS[c], _CHUNKS[c])],
                             out_hbm.at[pl.ds(base + _OFFS[c], _CHUNKS[c])], wsem)
        )
    for w in wrs:
        w.wait()


def kernel(t, embedding_weight):
    return _gather_kernel(t.astype(jnp.int32), embedding_weight)
